# baseline (speedup 1.0000x reference)
import functools

import jax
import jax.numpy as jnp
from jax import lax
from jax.experimental import pallas as pl
from jax.experimental.pallas import tpu as pltpu
from jax.experimental.pallas import tpu_sc as plsc

BUCKET_NUMBER = 1000
MIN_VALUE = 0.0
MAX_VALUE = 1.0
STEP = (MAX_VALUE - MIN_VALUE) / BUCKET_NUMBER
EMBED_DIM = 64

LANES = 16
BC = 128


def _make_kernel(BATCH, H, D):
    info = plsc.get_sparse_core_info()
    NC, NS = info.num_cores, info.num_subcores
    NW = NC * NS
    assert BATCH % (NW * BC) == 0 and H % 2 == 0 and D % LANES == 0
    rows_per_w = BATCH // NW
    n_blocks = rows_per_w // BC
    XB = BC * H

    mesh = plsc.VectorSubcoreMesh(core_axis_name="c", subcore_axis_name="s")

    @functools.partial(
        pl.kernel,
        out_type=jax.ShapeDtypeStruct((H, D, BATCH), jnp.float32),
        mesh=mesh,
        scratch_types=[
            pltpu.VMEM((XB,), jnp.float32),
            pltpu.VMEM((H, BC), jnp.int32),
            pltpu.VMEM((BC, D), jnp.float32),
            pltpu.VMEM((BC, D), jnp.float32),
            pltpu.VMEM((D, BC), jnp.float32),
            pltpu.VMEM((D, BC), jnp.float32),
            pltpu.SemaphoreType.DMA,
            pltpu.SemaphoreType.DMA,
            pltpu.SemaphoreType.DMA,
            pltpu.SemaphoreType.DMA,
        ],
        compiler_params=pltpu.CompilerParams(
            use_tc_tiling_on_sc=False, needs_layout_passes=False),
    )
    def k(x_hbm, table_hbm, out_hbm, x_v, idx_v,
          gbuf_a, gbuf_b, tbuf_a, tbuf_b, gsem_a, gsem_b, ssem_a, ssem_b):
        wid = lax.axis_index("s") * NC + lax.axis_index("c")
        w0 = wid * rows_per_w
        lane = lax.iota(jnp.int32, LANES)
        strided = lane * H
        gbufs, tbufs = (gbuf_a, gbuf_b), (tbuf_a, tbuf_b)
        gsems, ssems = (gsem_a, gsem_b), (ssem_a, ssem_b)

        def gather_h(h, kb):
            return pltpu.make_async_copy(
                table_hbm.at[idx_v.at[h]], gbufs[kb], gsems[kb])

        def store_h(h, b0, kb):
            return pltpu.make_async_copy(
                tbufs[kb], out_hbm.at[h, :, pl.ds(b0, BC)], ssems[kb])

        def transpose(kb):
            for d in range(D):
                col = jnp.full((LANES,), d, jnp.int32)
                for c in range(BC // LANES):
                    row = c * LANES + lane
                    v = plsc.load_gather(gbufs[kb], [row, col])
                    tbufs[kb][d, pl.ds(c * LANES, LANES)] = v

        def block_body(blk, carry):
            b0 = w0 + blk * BC
            pltpu.sync_copy(x_hbm.at[pl.ds(b0 * H, XB)], x_v)

            def idx_body(h, carry2):
                for c in range(BC // LANES):
                    xi = plsc.load_gather(x_v, [strided + (c * LANES * H + h)])
                    idx = ((xi - MIN_VALUE) / STEP).astype(jnp.int32)
                    idx = jnp.minimum(jnp.maximum(idx, 0), BUCKET_NUMBER - 1)
                    idx_v[h, pl.ds(c * LANES, LANES)] = idx
                return carry2

            lax.fori_loop(0, H, idx_body, 0)

            gather_h(0, 0).start()
            gather_h(1, 1).start()

            def h_body(j, carry2):
                for kb in range(2):
                    hh = 2 * j + kb
                    gather_h(hh, kb).wait()

                    @pl.when(j > 0)
                    def _(hh=hh, kb=kb):
                        store_h(hh - 2, b0, kb).wait()

                    transpose(kb)
                    store_h(hh, b0, kb).start()

                    @pl.when(j < H // 2 - 1)
                    def _(hh=hh, kb=kb):
                        gather_h(hh + 2, kb).start()
                return carry2

            lax.fori_loop(0, H // 2, h_body, 0)

            store_h(H - 2, b0, 0).wait()
            store_h(H - 1, b0, 1).wait()
            return carry

        lax.fori_loop(0, n_blocks, block_body, 0)

    return k


def kernel(x, table):
    if x.ndim == 2 and x.shape[1] == 1:
        x = jnp.squeeze(x, axis=-1)
    BATCH, H = x.shape
    D = table.shape[1]
    xf = x.reshape(BATCH * H)
    out_t = _make_kernel(BATCH, H, D)(xf, table)
    return jnp.transpose(out_t, (2, 0, 1))


# speedup vs baseline: 1.7045x; 1.7045x over previous
import functools

import jax
import jax.numpy as jnp
from jax import lax
from jax.experimental import pallas as pl
from jax.experimental.pallas import tpu as pltpu
from jax.experimental.pallas import tpu_sc as plsc

BUCKET_NUMBER = 1000
MIN_VALUE = 0.0
MAX_VALUE = 1.0
STEP = (MAX_VALUE - MIN_VALUE) / BUCKET_NUMBER
EMBED_DIM = 64

LANES = 16
BC = 128
LAT = 8


def _make_kernel(BATCH, H, D):
    info = plsc.get_sparse_core_info()
    NC, NS = info.num_cores, info.num_subcores
    NW = NC * NS
    assert BATCH % (NW * BC) == 0 and H % 2 == 0 and D % LAT == 0
    rows_per_w = BATCH // NW
    n_blocks = rows_per_w // BC
    XB = BC * H
    TSZ = BUCKET_NUMBER * D

    mesh = plsc.VectorSubcoreMesh(core_axis_name="c", subcore_axis_name="s")

    @functools.partial(
        pl.kernel,
        out_type=jax.ShapeDtypeStruct((H, D, BATCH), jnp.float32),
        mesh=mesh,
        scratch_types=[
            pltpu.VMEM((TSZ,), jnp.float32),
            pltpu.VMEM((XB,), jnp.float32),
            pltpu.VMEM((H, BC), jnp.int32),
            pltpu.VMEM((D, BC), jnp.float32),
            pltpu.VMEM((D, BC), jnp.float32),
            pltpu.SemaphoreType.DMA,
            pltpu.SemaphoreType.DMA,
        ],
        compiler_params=pltpu.CompilerParams(
            use_tc_tiling_on_sc=False, needs_layout_passes=False),
    )
    def k(xt_hbm, tab_hbm, out_hbm, tab_v, x_v, idx_v,
          tbuf_a, tbuf_b, ssem_a, ssem_b):
        wid = lax.axis_index("s") * NC + lax.axis_index("c")
        w0 = wid * rows_per_w
        lane = lax.iota(jnp.int32, LANES)
        strided = lane * H
        tbufs, ssems = (tbuf_a, tbuf_b), (ssem_a, ssem_b)

        pltpu.sync_copy(tab_hbm, tab_v)

        def store_h(h, b0, kb):
            return pltpu.make_async_copy(
                tbufs[kb], out_hbm.at[h, :, pl.ds(b0, BC)], ssems[kb])

        def block_body(blk, carry):
            b0 = w0 + blk * BC
            pltpu.sync_copy(xt_hbm.at[pl.ds(b0 * H, XB)], x_v)

            def idx_body(h, carry2):
                for c in range(BC // LANES):
                    xi = plsc.load_gather(x_v, [strided + (c * LANES * H + h)])
                    idx = ((xi - MIN_VALUE) / STEP).astype(jnp.int32)
                    idx = jnp.minimum(jnp.maximum(idx, 0), BUCKET_NUMBER - 1)
                    idx_v[h, pl.ds(c * LANES, LANES)] = idx
                return carry2

            lax.fori_loop(0, H, idx_body, 0)

            def h_body(j, carry2):
                for kb in range(2):
                    hh = 2 * j + kb

                    @pl.when(j > 0)
                    def _(hh=hh, kb=kb):
                        store_h(hh - 2, b0, kb).wait()

                    for c in range(BC // LANES):
                        iv = idx_v[hh, pl.ds(c * LANES, LANES)] * D
                        for db in range(0, D, LAT):
                            vs = [
                                plsc.load_gather(tab_v, [iv + ((db + q))])
                                for q in range(LAT)
                            ]
                            for q in range(LAT):
                                tbufs[kb][db + q, pl.ds(c * LANES, LANES)] = (
                                    vs[q])
                    store_h(hh, b0, kb).start()
                return carry2

            lax.fori_loop(0, H // 2, h_body, 0)

            store_h(H - 2, b0, 0).wait()
            store_h(H - 1, b0, 1).wait()
            return carry

        lax.fori_loop(0, n_blocks, block_body, 0)

    return k


def kernel(x, table):
    if x.ndim == 2 and x.shape[1] == 1:
        x = jnp.squeeze(x, axis=-1)
    BATCH, H = x.shape
    D = table.shape[1]
    xf = x.reshape(BATCH * H)
    tab = table.reshape(BUCKET_NUMBER * D)
    out_t = _make_kernel(BATCH, H, D)(xf, tab)
    return jnp.transpose(out_t, (2, 0, 1))


# speedup vs baseline: 3.0767x; 1.8050x over previous
import functools

import jax
import jax.numpy as jnp
from jax import lax
from jax.experimental import pallas as pl
from jax.experimental.pallas import tpu as pltpu
from jax.experimental.pallas import tpu_sc as plsc

BUCKET_NUMBER = 1000
MIN_VALUE = 0.0
MAX_VALUE = 1.0
STEP = (MAX_VALUE - MIN_VALUE) / BUCKET_NUMBER
EMBED_DIM = 64

LANES = 16
BC = 128
LAT = 8


def _make_kernel(BATCH, H, D):
    info = plsc.get_sparse_core_info()
    NC, NS = info.num_cores, info.num_subcores
    NW = NC * NS
    assert BATCH % (NW * BC) == 0 and H % 2 == 0 and D % LAT == 0
    rows_per_w = BATCH // NW
    n_blocks = rows_per_w // BC
    TSTRIDE = D + 1
    TSZ = BUCKET_NUMBER * TSTRIDE

    mesh = plsc.VectorSubcoreMesh(core_axis_name="c", subcore_axis_name="s")

    @functools.partial(
        pl.kernel,
        out_type=jax.ShapeDtypeStruct((H, D, BATCH), jnp.float32),
        mesh=mesh,
        scratch_types=[
            pltpu.VMEM((TSZ,), jnp.float32),
            pltpu.VMEM((H, BC), jnp.float32),
            pltpu.VMEM((H, BC), jnp.int32),
            pltpu.VMEM((D, BC), jnp.float32),
            pltpu.VMEM((D, BC), jnp.float32),
            pltpu.SemaphoreType.DMA,
            pltpu.SemaphoreType.DMA,
        ],
        compiler_params=pltpu.CompilerParams(
            use_tc_tiling_on_sc=False, needs_layout_passes=False),
    )
    def k(xt_hbm, tab_hbm, out_hbm, tab_v, x_v, idx_v,
          tbuf_a, tbuf_b, ssem_a, ssem_b):
        wid = lax.axis_index("s") * NC + lax.axis_index("c")
        w0 = wid * rows_per_w
        tbufs, ssems = (tbuf_a, tbuf_b), (ssem_a, ssem_b)

        pltpu.sync_copy(tab_hbm, tab_v)

        def store_h(h, b0, kb):
            return pltpu.make_async_copy(
                tbufs[kb], out_hbm.at[h, :, pl.ds(b0, BC)], ssems[kb])

        def block_body(blk, carry):
            b0 = w0 + blk * BC
            pltpu.sync_copy(xt_hbm.at[:, pl.ds(b0, BC)], x_v)

            def idx_body(h, carry2):
                for c in range(BC // LANES):
                    xi = x_v[h, pl.ds(c * LANES, LANES)]
                    idx = ((xi - MIN_VALUE) / STEP).astype(jnp.int32)
                    idx = jnp.minimum(jnp.maximum(idx, 0), BUCKET_NUMBER - 1)
                    idx_v[h, pl.ds(c * LANES, LANES)] = idx
                return carry2

            lax.fori_loop(0, H, idx_body, 0)

            def h_body(j, carry2):
                for kb in range(2):
                    hh = 2 * j + kb

                    @pl.when(j > 0)
                    def _(hh=hh, kb=kb):
                        store_h(hh - 2, b0, kb).wait()

                    for c in range(BC // LANES):
                        iv = idx_v[hh, pl.ds(c * LANES, LANES)] * TSTRIDE
                        for db in range(0, D, LAT):
                            vs = [
                                plsc.load_gather(tab_v, [iv + ((db + q))])
                                for q in range(LAT)
                            ]
                            for q in range(LAT):
                                tbufs[kb][db + q, pl.ds(c * LANES, LANES)] = (
                                    vs[q])
                    store_h(hh, b0, kb).start()
                return carry2

            lax.fori_loop(0, H // 2, h_body, 0)

            store_h(H - 2, b0, 0).wait()
            store_h(H - 1, b0, 1).wait()
            return carry

        lax.fori_loop(0, n_blocks, block_body, 0)

    return k


def kernel(x, table):
    if x.ndim == 2 and x.shape[1] == 1:
        x = jnp.squeeze(x, axis=-1)
    BATCH, H = x.shape
    D = table.shape[1]
    xt = jnp.transpose(x, (1, 0))
    tab = jnp.pad(table, ((0, 0), (0, 1))).reshape(BUCKET_NUMBER * (D + 1))
    out_t = _make_kernel(BATCH, H, D)(xt, tab)
    return jnp.transpose(out_t, (2, 0, 1))
